# baseline (speedup 1.0000x reference)
import functools

import jax
import jax.numpy as jnp
from jax import lax
from jax.experimental import pallas as pl
from jax.experimental.pallas import tpu as pltpu
from jax.experimental.pallas import tpu_sc as plsc

BATCH = 1024
VOCAB = 100000
NW = 32
ROWS_PER_TILE = BATCH // NW
CHUNK = VOCAB // 2
LANES = 16
BVR = 25
BLK = BVR * LANES
NBLK = CHUNK // BLK
BIG = 2 ** 30

_mesh = plsc.VectorSubcoreMesh(core_axis_name="c", subcore_axis_name="s")


@functools.partial(
    pl.kernel,
    mesh=_mesh,
    compiler_params=pltpu.CompilerParams(needs_layout_passes=False),
    out_type=(
        jax.ShapeDtypeStruct((BATCH,), jnp.float32),
        jax.ShapeDtypeStruct((BATCH,), jnp.float32),
    ),
    scratch_types=[
        pltpu.VMEM((CHUNK,), jnp.float32),
        pltpu.VMEM((CHUNK,), jnp.float32),
        pltpu.VMEM((NBLK * LANES,), jnp.float32),
        pltpu.VMEM((ROWS_PER_TILE,), jnp.int32),
        pltpu.VMEM((ROWS_PER_TILE,), jnp.float32),
        pltpu.VMEM((ROWS_PER_TILE,), jnp.float32),
        pltpu.SemaphoreType.DMA,
        pltpu.SemaphoreType.DMA,
        pltpu.SemaphoreType.DMA,
    ],
)
def _vocab_pp(inp_hbm, tab_hbm, cast_hbm, maxp_hbm,
              buf0, buf1, bmax, idx_v, maxp_v, cast_v, sem0, sem1, gsem):
    wid = lax.axis_index("s") * 2 + lax.axis_index("c")
    row0 = wid * ROWS_PER_TILE
    bufs = (buf0, buf1)
    sems = (sem0, sem1)
    lane = lax.broadcasted_iota(jnp.int32, (LANES,), 0)
    ninf = jnp.full((LANES,), -jnp.inf, jnp.float32)
    bigv = jnp.full((LANES,), BIG, jnp.int32)

    def dma(r, h):
        return pltpu.make_async_copy(
            inp_hbm.at[pl.ds((row0 + r) * VOCAB + h * CHUNK, CHUNK)],
            bufs[h], sems[h])

    dma(0, 0).start()
    dma(0, 1).start()

    def scan_chunk(bref):
        def blk(b, carry):
            base = b * BLK
            accs = [bref[pl.ds(base + i * LANES, LANES)] for i in range(4)]
            for u in range(4, BVR):
                v = bref[pl.ds(base + u * LANES, LANES)]
                accs[u % 4] = jnp.maximum(accs[u % 4], v)
            m01 = jnp.maximum(accs[0], accs[1])
            m23 = jnp.maximum(accs[2], accs[3])
            bmax[pl.ds(b * LANES, LANES)] = jnp.maximum(m01, m23)
            return carry

        lax.fori_loop(0, NBLK, blk, 0)

        def p2a(t, accs):
            a0, a1, a2, a3 = accs
            base = t * 5 * LANES
            a0 = jnp.maximum(a0, bmax[pl.ds(base, LANES)])
            a1 = jnp.maximum(a1, bmax[pl.ds(base + LANES, LANES)])
            a2 = jnp.maximum(a2, bmax[pl.ds(base + 2 * LANES, LANES)])
            a3 = jnp.maximum(a3, bmax[pl.ds(base + 3 * LANES, LANES)])
            a0 = jnp.maximum(a0, bmax[pl.ds(base + 4 * LANES, LANES)])
            return (a0, a1, a2, a3)

        a0, a1, a2, a3 = lax.fori_loop(0, NBLK // 5, p2a,
                                       (ninf, ninf, ninf, ninf))
        m = jnp.max(jnp.maximum(jnp.maximum(a0, a1), jnp.maximum(a2, a3)))

        def p2b(t, best):
            for u in range(5):
                b = t * 5 + u
                bm = bmax[pl.ds(b * LANES, LANES)]
                cand = jnp.where(bm == m, jnp.full((LANES,), b, jnp.int32),
                                 bigv)
                best = jnp.minimum(best, cand)
            return best

        bstar = jnp.min(lax.fori_loop(0, NBLK // 5, p2b, bigv))

        base = bstar * BLK

        def p2c(j, best):
            for u in range(5):
                off = base + (j * 5 + u) * LANES
                v = bref[pl.ds(off, LANES)]
                cand = jnp.where(v == m, lane + off, bigv)
                best = jnp.minimum(best, cand)
            return best

        i = jnp.min(lax.fori_loop(0, BVR // 5, p2c, bigv))
        return m, i

    def row_body(r, carry):
        rn = jnp.minimum(r + 1, ROWS_PER_TILE - 1)
        dma(r, 0).wait()
        m0, i0 = scan_chunk(buf0)
        dma(rn, 0).start()
        dma(r, 1).wait()
        m1, i1 = scan_chunk(buf1)
        dma(rn, 1).start()
        take1 = m1 > m0
        m = jnp.where(take1, m1, m0)
        i = jnp.where(take1, i1 + CHUNK, i0)
        rvec = jnp.full((LANES,), r, jnp.int32)
        msk0 = lane == 0
        plsc.store_scatter(maxp_v, [rvec], jnp.full((LANES,), m, jnp.float32),
                           mask=msk0)
        plsc.store_scatter(idx_v, [rvec], jnp.full((LANES,), i, jnp.int32),
                           mask=msk0)
        return carry

    lax.fori_loop(0, ROWS_PER_TILE, row_body, 0)
    dma(ROWS_PER_TILE - 1, 0).wait()
    dma(ROWS_PER_TILE - 1, 1).wait()

    g = pltpu.make_async_copy(tab_hbm.at[idx_v], cast_v, gsem)
    g.start()
    g.wait()
    pltpu.sync_copy(cast_v, cast_hbm.at[pl.ds(row0, ROWS_PER_TILE)])
    pltpu.sync_copy(maxp_v, maxp_hbm.at[pl.ds(row0, ROWS_PER_TILE)])


def kernel(input, table_values):
    return _vocab_pp(input.reshape(-1), table_values)


# speedup vs baseline: 1.8324x; 1.8324x over previous
import functools

import jax
import jax.numpy as jnp
from jax import lax
from jax.experimental import pallas as pl
from jax.experimental.pallas import tpu as pltpu
from jax.experimental.pallas import tpu_sc as plsc

BATCH = 1024
VOCAB = 100000
NW = 32
ROWS_PER_TILE = BATCH // NW
NGRP = ROWS_PER_TILE // 8
CCOLS = 4992
NCH = 20
BVR = 24
BLK = BVR * 16
NBLK = CCOLS // BLK
TAILW = 256
TAIL0 = VOCAB - TAILW
LANES = 16
BIG = 2 ** 30

_mesh = plsc.VectorSubcoreMesh(core_axis_name="c", subcore_axis_name="s")


@functools.partial(
    pl.kernel,
    mesh=_mesh,
    compiler_params=pltpu.CompilerParams(needs_layout_passes=False),
    out_type=(
        jax.ShapeDtypeStruct((BATCH,), jnp.float32),
        jax.ShapeDtypeStruct((BATCH,), jnp.float32),
    ),
    scratch_types=[
        pltpu.VMEM((8, CCOLS), jnp.float32),
        pltpu.VMEM((8, CCOLS), jnp.float32),
        pltpu.VMEM((8, TAILW), jnp.float32),
        pltpu.VMEM((8, NBLK * LANES), jnp.float32),
        pltpu.VMEM((ROWS_PER_TILE,), jnp.int32),
        pltpu.VMEM((ROWS_PER_TILE,), jnp.float32),
        pltpu.VMEM((ROWS_PER_TILE,), jnp.float32),
        pltpu.SMEM((8,), jnp.float32),
        pltpu.SMEM((8,), jnp.int32),
        pltpu.SemaphoreType.DMA,
        pltpu.SemaphoreType.DMA,
        pltpu.SemaphoreType.DMA,
        pltpu.SemaphoreType.DMA,
    ],
)
def _vocab_pp(inp_hbm, tail_hbm, tab_hbm, cast_hbm, maxp_hbm,
              buf0, buf1, tbuf, bmax, idx_v, maxp_v, cast_v,
              smem_m, smem_i, sem0, sem1, tsem, gsem):
    wid = lax.axis_index("s") * 2 + lax.axis_index("c")
    row0 = wid * ROWS_PER_TILE
    bufs = (buf0, buf1)
    sems = (sem0, sem1)
    lane = lax.broadcasted_iota(jnp.int32, (LANES,), 0)
    bigv = jnp.full((LANES,), BIG, jnp.int32)
    ninf = jnp.float32(-jnp.inf)

    def dma_chunk(g, c, par):
        rs = pl.multiple_of(row0 + g * 8, 8)
        cs = pl.multiple_of(c * CCOLS, 128)
        return pltpu.make_async_copy(
            inp_hbm.at[pl.ds(rs, 8), pl.ds(cs, CCOLS)], bufs[par], sems[par])

    def dma_tail(g):
        rs = pl.multiple_of(row0 + g * 8, 8)
        return pltpu.make_async_copy(
            tail_hbm.at[pl.ds(rs, 8), :], tbuf, tsem)

    def init_state(i, carry):
        smem_m[i] = ninf
        return carry

    lax.fori_loop(0, 8, init_state, 0)
    dma_chunk(0, 0, 0).start()
    dma_chunk(0, 1, 1).start()
    dma_tail(0).start()

    def vmax4(loads):
        accs = loads[:4]
        for u in range(4, len(loads)):
            accs[u % 4] = jnp.maximum(accs[u % 4], loads[u])
        return jnp.maximum(jnp.maximum(accs[0], accs[1]),
                           jnp.maximum(accs[2], accs[3]))

    def process_chunk(g, c, par):
        bref = bufs[par]

        def row_body(i, carry):
            def blk(b, carry2):
                base = b * BLK
                loads = [bref[i, pl.ds(base + u * LANES, LANES)]
                         for u in range(BVR)]
                bmax[i, pl.ds(b * LANES, LANES)] = vmax4(loads)
                return carry2

            lax.fori_loop(0, NBLK, blk, 0)
            m_c = jnp.max(vmax4([bmax[i, pl.ds(b * LANES, LANES)]
                                 for b in range(NBLK)]))

            @pl.when(m_c > smem_m[i])
            def _():
                bmin = bigv
                for b in range(NBLK):
                    bm = bmax[i, pl.ds(b * LANES, LANES)]
                    bmin = jnp.minimum(
                        bmin, jnp.where(bm == m_c,
                                        jnp.full((LANES,), b, jnp.int32),
                                        bigv))
                bstar = jnp.min(bmin)
                base = bstar * BLK
                gbase = lane + (c * CCOLS + base)
                best = bigv
                for u in range(BVR):
                    v = bref[i, pl.ds(base + u * LANES, LANES)]
                    best = jnp.minimum(
                        best, jnp.where(v == m_c, gbase + u * LANES, bigv))
                smem_m[i] = m_c
                smem_i[i] = jnp.min(best)

            return carry

        lax.fori_loop(0, 8, row_body, 0)

    for g in range(NGRP):
        def chunk_pair(cc, carry, g=g):
            for par in (0, 1):
                c = cc * 2 + par
                dma_chunk(g, c, par).wait()
                process_chunk(g, c, par)

                @pl.when(c + 2 < NCH)
                def _(c=c, par=par, g=g):
                    dma_chunk(g, c + 2, par).start()

            return carry

        lax.fori_loop(0, NCH // 2, chunk_pair, 0)

        dma_tail(g).wait()

        def tail_row(i, carry):
            m_t = jnp.max(vmax4([tbuf[i, pl.ds(u * LANES, LANES)]
                                 for u in range(TAILW // LANES)]))

            @pl.when(m_t > smem_m[i])
            def _():
                best = bigv
                gbase = lane + TAIL0
                for u in range(TAILW // LANES):
                    v = tbuf[i, pl.ds(u * LANES, LANES)]
                    best = jnp.minimum(
                        best, jnp.where(v == m_t, gbase + u * LANES, bigv))
                smem_m[i] = m_t
                smem_i[i] = jnp.min(best)

            return carry

        lax.fori_loop(0, 8, tail_row, 0)

        def flush_row(i, carry, g=g):
            rvec = jnp.full((LANES,), g * 8 + i, jnp.int32)
            msk0 = lane == 0
            plsc.store_scatter(maxp_v, [rvec],
                               jnp.full((LANES,), smem_m[i], jnp.float32),
                               mask=msk0)
            plsc.store_scatter(idx_v, [rvec],
                               jnp.full((LANES,), smem_i[i], jnp.int32),
                               mask=msk0)
            smem_m[i] = ninf
            return carry

        lax.fori_loop(0, 8, flush_row, 0)

        if g + 1 < NGRP:
            dma_chunk(g + 1, 0, 0).start()
            dma_chunk(g + 1, 1, 1).start()
            dma_tail(g + 1).start()

    gth = pltpu.make_async_copy(tab_hbm.at[idx_v], cast_v, gsem)
    gth.start()
    gth.wait()
    pltpu.sync_copy(cast_v, cast_hbm.at[pl.ds(row0, ROWS_PER_TILE)])
    pltpu.sync_copy(maxp_v, maxp_hbm.at[pl.ds(row0, ROWS_PER_TILE)])


def kernel(input, table_values):
    tail = lax.slice(input, (0, TAIL0), (BATCH, VOCAB))
    return _vocab_pp(input, tail, table_values)


# speedup vs baseline: 5.4069x; 2.9507x over previous
import functools

import jax
import jax.numpy as jnp
from jax import lax
from jax.experimental import pallas as pl
from jax.experimental.pallas import tpu as pltpu
from jax.experimental.pallas import tpu_sc as plsc

BATCH = 1024
VOCAB = 100000
NQ = 4
NCOL = 8
QV = VOCAB // NQ
VC = 200
NCH = QV // VC
LANES = 16
JV = 128 // LANES

_mesh = plsc.VectorSubcoreMesh(core_axis_name="c", subcore_axis_name="s")


@functools.partial(
    pl.kernel,
    mesh=_mesh,
    compiler_params=pltpu.CompilerParams(needs_layout_passes=False),
    out_type=(
        jax.ShapeDtypeStruct((BATCH,), jnp.float32),
        jax.ShapeDtypeStruct((BATCH,), jnp.float32),
    ),
    scratch_types=[
        pltpu.VMEM((VC, 128), jnp.float32),
        pltpu.VMEM((VC, 128), jnp.float32),
        pltpu.VMEM((128,), jnp.float32),
        pltpu.VMEM((128,), jnp.int32),
        pltpu.VMEM((128,), jnp.float32),
        pltpu.VMEM((128,), jnp.int32),
        pltpu.VMEM((128,), jnp.float32),
        pltpu.VMEM_SHARED((16, 128), jnp.float32),
        pltpu.VMEM_SHARED((16, 128), jnp.int32),
        pltpu.SemaphoreType.DMA,
        pltpu.SemaphoreType.DMA,
        pltpu.SemaphoreType.DMA,
    ],
)
def _vocab_pp(inpt_hbm, tab_hbm, cast_hbm, maxp_hbm,
              buf0, buf1, m_v, i_v, tm_v, ti_v, cast_v,
              shm_m, shm_i, sem0, sem1, gsem):
    cid = lax.axis_index("c")
    sid = lax.axis_index("s")
    col = cid * 4 + sid % 4
    q = sid // 4
    b0 = col * 128
    v0 = q * QV
    bufs = (buf0, buf1)
    sems = (sem0, sem1)
    ninf = jnp.full((LANES,), -jnp.inf, jnp.float32)
    zero = jnp.zeros((LANES,), jnp.int32)

    def dma(c, par):
        vs = pl.multiple_of(v0 + c * VC, 8)
        bs = pl.multiple_of(b0, 128)
        return pltpu.make_async_copy(
            inpt_hbm.at[pl.ds(vs, VC), pl.ds(bs, 128)], bufs[par], sems[par])

    dma(0, 0).start()
    dma(1, 1).start()

    def scan_chunk(c, par, acc):
        bref = bufs[par]

        def vrow(t, acc2):
            accl = list(acc2)
            for k in range(2):
                v = t * 2 + k
                gv = v0 + c * VC + v
                gvv = jnp.full((LANES,), gv, jnp.int32)
                for j in range(JV):
                    x = bref[v, pl.ds(j * LANES, LANES)]
                    better = x > accl[j]
                    accl[j] = jnp.where(better, x, accl[j])
                    accl[JV + j] = jnp.where(better, gvv, accl[JV + j])
            return tuple(accl)

        return lax.fori_loop(0, VC // 2, vrow, acc)

    def chunk_pair(cc, acc):
        for par in (0, 1):
            c = cc * 2 + par
            dma(c, par).wait()
            acc = scan_chunk(c, par, acc)

            @pl.when(c + 2 < NCH)
            def _(c=c, par=par):
                dma(c + 2, par).start()

        return acc

    acc = lax.fori_loop(0, (NCH - 1) // 2, chunk_pair,
                        (ninf,) * JV + (zero,) * JV)
    dma(NCH - 1, 0).wait()
    acc = scan_chunk(NCH - 1, 0, acc)

    for j in range(JV):
        m_v[pl.ds(j * LANES, LANES)] = acc[j]
        i_v[pl.ds(j * LANES, LANES)] = acc[JV + j]
    pltpu.sync_copy(m_v, shm_m.at[sid])
    pltpu.sync_copy(i_v, shm_i.at[sid])
    plsc.subcore_barrier()

    @pl.when(sid < 4)
    def _():
        for p in range(1, NQ):
            pltpu.sync_copy(shm_m.at[sid + p * 4], tm_v)
            pltpu.sync_copy(shm_i.at[sid + p * 4], ti_v)
            for j in range(JV):
                sl = pl.ds(j * LANES, LANES)
                m1, i1 = m_v[sl], i_v[sl]
                m2, i2 = tm_v[sl], ti_v[sl]
                take2 = (m2 > m1) | ((m2 == m1) & (i2 < i1))
                m_v[sl] = jnp.where(take2, m2, m1)
                i_v[sl] = jnp.where(take2, i2, i1)
        g = pltpu.make_async_copy(tab_hbm.at[i_v], cast_v, gsem)
        g.start()
        g.wait()
        bs = pl.multiple_of(b0, 128)
        pltpu.sync_copy(cast_v, cast_hbm.at[pl.ds(bs, 128)])
        pltpu.sync_copy(m_v, maxp_hbm.at[pl.ds(bs, 128)])


def kernel(input, table_values):
    return _vocab_pp(input.T, table_values)


# speedup vs baseline: 6.7897x; 1.2558x over previous
import functools

import jax
import jax.numpy as jnp
from jax import lax
from jax.experimental import pallas as pl
from jax.experimental.pallas import tpu as pltpu
from jax.experimental.pallas import tpu_sc as plsc

BATCH = 1024
VOCAB = 100000
VS = 52000
NQ = 4
QV = VS // NQ
VC = 200
NCH = QV // VC
LANES = 16
JV = 128 // LANES
BV = 1000
TCN = (VOCAB - VS) // BV

_mesh = plsc.VectorSubcoreMesh(core_axis_name="c", subcore_axis_name="s")


@functools.partial(
    pl.kernel,
    mesh=_mesh,
    compiler_params=pltpu.CompilerParams(needs_layout_passes=False),
    out_type=(
        jax.ShapeDtypeStruct((BATCH,), jnp.float32),
        jax.ShapeDtypeStruct((BATCH,), jnp.int32),
    ),
    scratch_types=[
        pltpu.VMEM((VC, 128), jnp.float32),
        pltpu.VMEM((VC, 128), jnp.float32),
        pltpu.VMEM((128,), jnp.float32),
        pltpu.VMEM((128,), jnp.int32),
        pltpu.VMEM((128,), jnp.float32),
        pltpu.VMEM((128,), jnp.int32),
        pltpu.VMEM_SHARED((16, 128), jnp.float32),
        pltpu.VMEM_SHARED((16, 128), jnp.int32),
        pltpu.SemaphoreType.DMA,
        pltpu.SemaphoreType.DMA,
    ],
)
def _sc_scan(inpt_hbm, pm_hbm, pi_hbm,
             buf0, buf1, m_v, i_v, tm_v, ti_v,
             shm_m, shm_i, sem0, sem1):
    cid = lax.axis_index("c")
    sid = lax.axis_index("s")
    col = cid * 4 + sid % 4
    q = sid // 4
    b0 = col * 128
    v0 = q * QV
    bufs = (buf0, buf1)
    sems = (sem0, sem1)
    ninf = jnp.full((LANES,), -jnp.inf, jnp.float32)
    zero = jnp.zeros((LANES,), jnp.int32)

    def dma(c, par):
        vs = pl.multiple_of(v0 + c * VC, 8)
        bs = pl.multiple_of(b0, 128)
        return pltpu.make_async_copy(
            inpt_hbm.at[pl.ds(vs, VC), pl.ds(bs, 128)], bufs[par], sems[par])

    dma(0, 0).start()
    dma(1, 1).start()

    def scan_chunk(c, par, acc):
        bref = bufs[par]

        def vrow(t, acc2):
            accl = list(acc2)
            for k in range(2):
                v = t * 2 + k
                gv = v0 + c * VC + v
                gvv = jnp.full((LANES,), gv, jnp.int32)
                for j in range(JV):
                    x = bref[v, pl.ds(j * LANES, LANES)]
                    better = x > accl[j]
                    accl[j] = jnp.where(better, x, accl[j])
                    accl[JV + j] = jnp.where(better, gvv, accl[JV + j])
            return tuple(accl)

        return lax.fori_loop(0, VC // 2, vrow, acc)

    def chunk_pair(cc, acc):
        for par in (0, 1):
            c = cc * 2 + par
            dma(c, par).wait()
            acc = scan_chunk(c, par, acc)

            @pl.when(c + 2 < NCH)
            def _(c=c, par=par):
                dma(c + 2, par).start()

        return acc

    acc = lax.fori_loop(0, (NCH - 1) // 2, chunk_pair,
                        (ninf,) * JV + (zero,) * JV)
    dma(NCH - 1, 0).wait()
    acc = scan_chunk(NCH - 1, 0, acc)

    for j in range(JV):
        m_v[pl.ds(j * LANES, LANES)] = acc[j]
        i_v[pl.ds(j * LANES, LANES)] = acc[JV + j]
    pltpu.sync_copy(m_v, shm_m.at[sid])
    pltpu.sync_copy(i_v, shm_i.at[sid])
    plsc.subcore_barrier()

    @pl.when(sid < 4)
    def _():
        for p in range(1, NQ):
            pltpu.sync_copy(shm_m.at[sid + p * 4], tm_v)
            pltpu.sync_copy(shm_i.at[sid + p * 4], ti_v)
            for j in range(JV):
                sl = pl.ds(j * LANES, LANES)
                m1, i1 = m_v[sl], i_v[sl]
                m2, i2 = tm_v[sl], ti_v[sl]
                take2 = (m2 > m1) | ((m2 == m1) & (i2 < i1))
                m_v[sl] = jnp.where(take2, m2, m1)
                i_v[sl] = jnp.where(take2, i2, i1)
        bs = pl.multiple_of(b0, 128)
        pltpu.sync_copy(m_v, pm_hbm.at[pl.ds(bs, 128)])
        pltpu.sync_copy(i_v, pi_hbm.at[pl.ds(bs, 128)])


def _tc_body(x_ref, om_ref, oi_ref, rm, ri):
    step = pl.program_id(0)

    @pl.when(step == 0)
    def _():
        rm[...] = jnp.full((8, BATCH), -jnp.inf, jnp.float32)
        ri[...] = jnp.zeros((8, BATCH), jnp.int32)

    iota8 = lax.broadcasted_iota(jnp.int32, (8, BATCH), 0)
    m = rm[...]
    idx = ri[...]
    for k in range(BV // 8):
        x = x_ref[pl.ds(k * 8, 8), :]
        gv = iota8 + (VS + step * BV + k * 8)
        better = x > m
        m = jnp.where(better, x, m)
        idx = jnp.where(better, gv, idx)
    rm[...] = m
    ri[...] = idx
    om_ref[...] = m
    oi_ref[...] = idx


_tc_scan = pl.pallas_call(
    _tc_body,
    grid=(TCN,),
    in_specs=[pl.BlockSpec((BV, BATCH), lambda i: (i + VS // BV, 0))],
    out_specs=(pl.BlockSpec((8, BATCH), lambda i: (0, 0)),
               pl.BlockSpec((8, BATCH), lambda i: (0, 0))),
    out_shape=(jax.ShapeDtypeStruct((8, BATCH), jnp.float32),
               jax.ShapeDtypeStruct((8, BATCH), jnp.int32)),
    scratch_shapes=[pltpu.VMEM((8, BATCH), jnp.float32),
                    pltpu.VMEM((8, BATCH), jnp.int32)],
)


@functools.partial(
    pl.kernel,
    mesh=_mesh,
    compiler_params=pltpu.CompilerParams(needs_layout_passes=False),
    out_type=(
        jax.ShapeDtypeStruct((BATCH,), jnp.float32),
        jax.ShapeDtypeStruct((BATCH,), jnp.float32),
    ),
    scratch_types=[
        pltpu.VMEM((128,), jnp.float32),
        pltpu.VMEM((128,), jnp.int32),
        pltpu.VMEM((8, 128), jnp.float32),
        pltpu.VMEM((8, 128), jnp.int32),
        pltpu.VMEM((128,), jnp.float32),
        pltpu.SemaphoreType.DMA,
    ],
)
def _merge(pm_hbm, pi_hbm, tcm_hbm, tci_hbm, tab_hbm, cast_hbm, maxp_hbm,
           m_v, i_v, tm_v, ti_v, cast_v, gsem):
    cid = lax.axis_index("c")
    sid = lax.axis_index("s")

    @pl.when(sid < 4)
    def _():
        col = cid * 4 + sid
        bs = pl.multiple_of(col * 128, 128)
        pltpu.sync_copy(pm_hbm.at[pl.ds(bs, 128)], m_v)
        pltpu.sync_copy(pi_hbm.at[pl.ds(bs, 128)], i_v)
        pltpu.sync_copy(tcm_hbm.at[:, pl.ds(bs, 128)], tm_v)
        pltpu.sync_copy(tci_hbm.at[:, pl.ds(bs, 128)], ti_v)
        for j in range(JV):
            sl = pl.ds(j * LANES, LANES)
            m1, i1 = m_v[sl], i_v[sl]
            for r in range(8):
                m2 = tm_v[r, sl]
                i2 = ti_v[r, sl]
                take2 = (m2 > m1) | ((m2 == m1) & (i2 < i1))
                m1 = jnp.where(take2, m2, m1)
                i1 = jnp.where(take2, i2, i1)
            m_v[sl] = m1
            i_v[sl] = i1
        g = pltpu.make_async_copy(tab_hbm.at[i_v], cast_v, gsem)
        g.start()
        g.wait()
        pltpu.sync_copy(cast_v, cast_hbm.at[pl.ds(bs, 128)])
        pltpu.sync_copy(m_v, maxp_hbm.at[pl.ds(bs, 128)])


def kernel(input, table_values):
    inpt = input.T
    sm, si = _sc_scan(inpt)
    tm, ti = _tc_scan(inpt)
    return _merge(sm, si, tm, ti, table_values)


# speedup vs baseline: 6.9501x; 1.0236x over previous
import functools

import jax
import jax.numpy as jnp
from jax import lax
from jax.experimental import pallas as pl
from jax.experimental.pallas import tpu as pltpu
from jax.experimental.pallas import tpu_sc as plsc

BATCH = 1024
VOCAB = 100000
VS = 50400
NQ = 4
QV = VS // NQ
VC = 200
NCH = QV // VC
LANES = 16
JV = 128 // LANES
BV = 800
TCN = (VOCAB - VS) // BV

_mesh = plsc.VectorSubcoreMesh(core_axis_name="c", subcore_axis_name="s")


@functools.partial(
    pl.kernel,
    mesh=_mesh,
    compiler_params=pltpu.CompilerParams(needs_layout_passes=False),
    out_type=(
        jax.ShapeDtypeStruct((BATCH,), jnp.float32),
        jax.ShapeDtypeStruct((BATCH,), jnp.int32),
    ),
    scratch_types=[
        pltpu.VMEM((VC, 128), jnp.float32),
        pltpu.VMEM((VC, 128), jnp.float32),
        pltpu.VMEM((128,), jnp.float32),
        pltpu.VMEM((128,), jnp.int32),
        pltpu.VMEM((128,), jnp.float32),
        pltpu.VMEM((128,), jnp.int32),
        pltpu.VMEM_SHARED((16, 128), jnp.float32),
        pltpu.VMEM_SHARED((16, 128), jnp.int32),
        pltpu.SemaphoreType.DMA,
        pltpu.SemaphoreType.DMA,
    ],
)
def _sc_scan(inpt_hbm, pm_hbm, pi_hbm,
             buf0, buf1, m_v, i_v, tm_v, ti_v,
             shm_m, shm_i, sem0, sem1):
    cid = lax.axis_index("c")
    sid = lax.axis_index("s")
    col = cid * 4 + sid % 4
    q = sid // 4
    b0 = col * 128
    v0 = q * QV
    bufs = (buf0, buf1)
    sems = (sem0, sem1)
    ninf = jnp.full((LANES,), -jnp.inf, jnp.float32)
    zero = jnp.zeros((LANES,), jnp.int32)

    def dma(c, par):
        vs = pl.multiple_of(v0 + c * VC, 8)
        bs = pl.multiple_of(b0, 128)
        return pltpu.make_async_copy(
            inpt_hbm.at[pl.ds(vs, VC), pl.ds(bs, 128)], bufs[par], sems[par])

    dma(0, 0).start()
    dma(1, 1).start()

    def scan_chunk(c, par, acc):
        bref = bufs[par]

        def vrow(t, acc2):
            accl = list(acc2)
            for k in range(2):
                v = t * 2 + k
                gv = v0 + c * VC + v
                gvv = jnp.full((LANES,), gv, jnp.int32)
                for j in range(JV):
                    x = bref[v, pl.ds(j * LANES, LANES)]
                    better = x > accl[j]
                    accl[j] = jnp.where(better, x, accl[j])
                    accl[JV + j] = jnp.where(better, gvv, accl[JV + j])
            return tuple(accl)

        return lax.fori_loop(0, VC // 2, vrow, acc)

    def chunk_pair(cc, acc):
        for par in (0, 1):
            c = cc * 2 + par
            dma(c, par).wait()
            acc = scan_chunk(c, par, acc)

            @pl.when(c + 2 < NCH)
            def _(c=c, par=par):
                dma(c + 2, par).start()

        return acc

    acc = lax.fori_loop(0, (NCH - 1) // 2, chunk_pair,
                        (ninf,) * JV + (zero,) * JV)
    dma(NCH - 1, 0).wait()
    acc = scan_chunk(NCH - 1, 0, acc)

    for j in range(JV):
        m_v[pl.ds(j * LANES, LANES)] = acc[j]
        i_v[pl.ds(j * LANES, LANES)] = acc[JV + j]
    pltpu.sync_copy(m_v, shm_m.at[sid])
    pltpu.sync_copy(i_v, shm_i.at[sid])
    plsc.subcore_barrier()

    @pl.when(sid < 4)
    def _():
        for p in range(1, NQ):
            pltpu.sync_copy(shm_m.at[sid + p * 4], tm_v)
            pltpu.sync_copy(shm_i.at[sid + p * 4], ti_v)
            for j in range(JV):
                sl = pl.ds(j * LANES, LANES)
                m1, i1 = m_v[sl], i_v[sl]
                m2, i2 = tm_v[sl], ti_v[sl]
                take2 = (m2 > m1) | ((m2 == m1) & (i2 < i1))
                m_v[sl] = jnp.where(take2, m2, m1)
                i_v[sl] = jnp.where(take2, i2, i1)
        bs = pl.multiple_of(b0, 128)
        pltpu.sync_copy(m_v, pm_hbm.at[pl.ds(bs, 128)])
        pltpu.sync_copy(i_v, pi_hbm.at[pl.ds(bs, 128)])


def _tc_body(x_ref, om_ref, oi_ref, rm, ri):
    step = pl.program_id(0)

    @pl.when(step == 0)
    def _():
        rm[...] = jnp.full((8, BATCH), -jnp.inf, jnp.float32)
        ri[...] = jnp.zeros((8, BATCH), jnp.int32)

    iota8 = lax.broadcasted_iota(jnp.int32, (8, BATCH), 0)
    m = rm[...]
    idx = ri[...]
    for k in range(BV // 8):
        x = x_ref[pl.ds(k * 8, 8), :]
        gv = iota8 + (VS + step * BV + k * 8)
        better = x > m
        m = jnp.where(better, x, m)
        idx = jnp.where(better, gv, idx)
    rm[...] = m
    ri[...] = idx
    om_ref[...] = m
    oi_ref[...] = idx


_tc_scan = pl.pallas_call(
    _tc_body,
    grid=(TCN,),
    in_specs=[pl.BlockSpec((BV, BATCH), lambda i: (i + VS // BV, 0))],
    out_specs=(pl.BlockSpec((8, BATCH), lambda i: (0, 0)),
               pl.BlockSpec((8, BATCH), lambda i: (0, 0))),
    out_shape=(jax.ShapeDtypeStruct((8, BATCH), jnp.float32),
               jax.ShapeDtypeStruct((8, BATCH), jnp.int32)),
    scratch_shapes=[pltpu.VMEM((8, BATCH), jnp.float32),
                    pltpu.VMEM((8, BATCH), jnp.int32)],
)


@functools.partial(
    pl.kernel,
    mesh=_mesh,
    compiler_params=pltpu.CompilerParams(needs_layout_passes=False),
    out_type=(
        jax.ShapeDtypeStruct((BATCH,), jnp.float32),
        jax.ShapeDtypeStruct((BATCH,), jnp.float32),
    ),
    scratch_types=[
        pltpu.VMEM((128,), jnp.float32),
        pltpu.VMEM((128,), jnp.int32),
        pltpu.VMEM((8, 128), jnp.float32),
        pltpu.VMEM((8, 128), jnp.int32),
        pltpu.VMEM((128,), jnp.float32),
        pltpu.SemaphoreType.DMA,
    ],
)
def _merge(pm_hbm, pi_hbm, tcm_hbm, tci_hbm, tab_hbm, cast_hbm, maxp_hbm,
           m_v, i_v, tm_v, ti_v, cast_v, gsem):
    cid = lax.axis_index("c")
    sid = lax.axis_index("s")

    @pl.when(sid < 4)
    def _():
        col = cid * 4 + sid
        bs = pl.multiple_of(col * 128, 128)
        pltpu.sync_copy(pm_hbm.at[pl.ds(bs, 128)], m_v)
        pltpu.sync_copy(pi_hbm.at[pl.ds(bs, 128)], i_v)
        pltpu.sync_copy(tcm_hbm.at[:, pl.ds(bs, 128)], tm_v)
        pltpu.sync_copy(tci_hbm.at[:, pl.ds(bs, 128)], ti_v)
        for j in range(JV):
            sl = pl.ds(j * LANES, LANES)
            m1, i1 = m_v[sl], i_v[sl]
            for r in range(8):
                m2 = tm_v[r, sl]
                i2 = ti_v[r, sl]
                take2 = (m2 > m1) | ((m2 == m1) & (i2 < i1))
                m1 = jnp.where(take2, m2, m1)
                i1 = jnp.where(take2, i2, i1)
            m_v[sl] = m1
            i_v[sl] = i1
        g = pltpu.make_async_copy(tab_hbm.at[i_v], cast_v, gsem)
        g.start()
        g.wait()
        pltpu.sync_copy(cast_v, cast_hbm.at[pl.ds(bs, 128)])
        pltpu.sync_copy(m_v, maxp_hbm.at[pl.ds(bs, 128)])


def kernel(input, table_values):
    inpt = input.T
    sm, si = _sc_scan(inpt)
    tm, ti = _tc_scan(inpt)
    return _merge(sm, si, tm, ti, table_values)


# speedup vs baseline: 6.9832x; 1.0048x over previous
import functools

import jax
import jax.numpy as jnp
from jax import lax
from jax.experimental import pallas as pl
from jax.experimental.pallas import tpu as pltpu
from jax.experimental.pallas import tpu_sc as plsc

BATCH = 1024
VOCAB = 100000
VS = 48000
NQ = 4
QV = VS // NQ
VC = 200
NCH = QV // VC
LANES = 16
JV = 128 // LANES
BV = 2000
TCN = (VOCAB - VS) // BV

_mesh = plsc.VectorSubcoreMesh(core_axis_name="c", subcore_axis_name="s")


@functools.partial(
    pl.kernel,
    mesh=_mesh,
    compiler_params=pltpu.CompilerParams(needs_layout_passes=False),
    out_type=(
        jax.ShapeDtypeStruct((BATCH,), jnp.float32),
        jax.ShapeDtypeStruct((BATCH,), jnp.int32),
    ),
    scratch_types=[
        pltpu.VMEM((VC, 128), jnp.float32),
        pltpu.VMEM((VC, 128), jnp.float32),
        pltpu.VMEM((128,), jnp.float32),
        pltpu.VMEM((128,), jnp.int32),
        pltpu.VMEM((128,), jnp.float32),
        pltpu.VMEM((128,), jnp.int32),
        pltpu.VMEM_SHARED((16, 128), jnp.float32),
        pltpu.VMEM_SHARED((16, 128), jnp.int32),
        pltpu.SemaphoreType.DMA,
        pltpu.SemaphoreType.DMA,
    ],
)
def _sc_scan(inpt_hbm, pm_hbm, pi_hbm,
             buf0, buf1, m_v, i_v, tm_v, ti_v,
             shm_m, shm_i, sem0, sem1):
    cid = lax.axis_index("c")
    sid = lax.axis_index("s")
    col = cid * 4 + sid % 4
    q = sid // 4
    b0 = col * 128
    v0 = q * QV
    bufs = (buf0, buf1)
    sems = (sem0, sem1)
    ninf = jnp.full((LANES,), -jnp.inf, jnp.float32)
    zero = jnp.zeros((LANES,), jnp.int32)

    def dma(c, par):
        vs = pl.multiple_of(v0 + c * VC, 8)
        bs = pl.multiple_of(b0, 128)
        return pltpu.make_async_copy(
            inpt_hbm.at[pl.ds(vs, VC), pl.ds(bs, 128)], bufs[par], sems[par])

    dma(0, 0).start()
    dma(1, 1).start()

    def scan_chunk(c, par, acc):
        bref = bufs[par]

        def vrow(t, acc2):
            accl = list(acc2)
            for k in range(2):
                v = t * 2 + k
                gv = v0 + c * VC + v
                gvv = jnp.full((LANES,), gv, jnp.int32)
                for j in range(JV):
                    x = bref[v, pl.ds(j * LANES, LANES)]
                    better = x > accl[j]
                    accl[j] = jnp.where(better, x, accl[j])
                    accl[JV + j] = jnp.where(better, gvv, accl[JV + j])
            return tuple(accl)

        return lax.fori_loop(0, VC // 2, vrow, acc)

    def chunk_pair(cc, acc):
        for par in (0, 1):
            c = cc * 2 + par
            dma(c, par).wait()
            acc = scan_chunk(c, par, acc)

            @pl.when(c + 2 < NCH)
            def _(c=c, par=par):
                dma(c + 2, par).start()

        return acc

    acc = lax.fori_loop(0, NCH // 2, chunk_pair,
                        (ninf,) * JV + (zero,) * JV)
    if NCH % 2 == 1:
        dma(NCH - 1, 0).wait()
        acc = scan_chunk(NCH - 1, 0, acc)

    for j in range(JV):
        m_v[pl.ds(j * LANES, LANES)] = acc[j]
        i_v[pl.ds(j * LANES, LANES)] = acc[JV + j]
    pltpu.sync_copy(m_v, shm_m.at[sid])
    pltpu.sync_copy(i_v, shm_i.at[sid])
    plsc.subcore_barrier()

    @pl.when(sid < 4)
    def _():
        for p in range(1, NQ):
            pltpu.sync_copy(shm_m.at[sid + p * 4], tm_v)
            pltpu.sync_copy(shm_i.at[sid + p * 4], ti_v)
            for j in range(JV):
                sl = pl.ds(j * LANES, LANES)
                m1, i1 = m_v[sl], i_v[sl]
                m2, i2 = tm_v[sl], ti_v[sl]
                take2 = (m2 > m1) | ((m2 == m1) & (i2 < i1))
                m_v[sl] = jnp.where(take2, m2, m1)
                i_v[sl] = jnp.where(take2, i2, i1)
        bs = pl.multiple_of(b0, 128)
        pltpu.sync_copy(m_v, pm_hbm.at[pl.ds(bs, 128)])
        pltpu.sync_copy(i_v, pi_hbm.at[pl.ds(bs, 128)])


def _tc_body(x_ref, om_ref, oi_ref, rm, ri):
    step = pl.program_id(0)

    @pl.when(step == 0)
    def _():
        rm[...] = jnp.full((8, BATCH), -jnp.inf, jnp.float32)
        ri[...] = jnp.zeros((8, BATCH), jnp.int32)

    iota8 = lax.broadcasted_iota(jnp.int32, (8, BATCH), 0)
    m = rm[...]
    idx = ri[...]
    for k in range(BV // 8):
        x = x_ref[pl.ds(k * 8, 8), :]
        gv = iota8 + (VS + step * BV + k * 8)
        better = x > m
        m = jnp.where(better, x, m)
        idx = jnp.where(better, gv, idx)
    rm[...] = m
    ri[...] = idx
    om_ref[...] = m
    oi_ref[...] = idx


_tc_scan = pl.pallas_call(
    _tc_body,
    grid=(TCN,),
    in_specs=[pl.BlockSpec((BV, BATCH), lambda i: (i + VS // BV, 0))],
    out_specs=(pl.BlockSpec((8, BATCH), lambda i: (0, 0)),
               pl.BlockSpec((8, BATCH), lambda i: (0, 0))),
    out_shape=(jax.ShapeDtypeStruct((8, BATCH), jnp.float32),
               jax.ShapeDtypeStruct((8, BATCH), jnp.int32)),
    scratch_shapes=[pltpu.VMEM((8, BATCH), jnp.float32),
                    pltpu.VMEM((8, BATCH), jnp.int32)],
)


@functools.partial(
    pl.kernel,
    mesh=_mesh,
    compiler_params=pltpu.CompilerParams(needs_layout_passes=False),
    out_type=(
        jax.ShapeDtypeStruct((BATCH,), jnp.float32),
        jax.ShapeDtypeStruct((BATCH,), jnp.float32),
    ),
    scratch_types=[
        pltpu.VMEM((128,), jnp.float32),
        pltpu.VMEM((128,), jnp.int32),
        pltpu.VMEM((8, 128), jnp.float32),
        pltpu.VMEM((8, 128), jnp.int32),
        pltpu.VMEM((128,), jnp.float32),
        pltpu.SemaphoreType.DMA,
    ],
)
def _merge(pm_hbm, pi_hbm, tcm_hbm, tci_hbm, tab_hbm, cast_hbm, maxp_hbm,
           m_v, i_v, tm_v, ti_v, cast_v, gsem):
    cid = lax.axis_index("c")
    sid = lax.axis_index("s")

    @pl.when(sid < 4)
    def _():
        col = cid * 4 + sid
        bs = pl.multiple_of(col * 128, 128)
        pltpu.sync_copy(pm_hbm.at[pl.ds(bs, 128)], m_v)
        pltpu.sync_copy(pi_hbm.at[pl.ds(bs, 128)], i_v)
        pltpu.sync_copy(tcm_hbm.at[:, pl.ds(bs, 128)], tm_v)
        pltpu.sync_copy(tci_hbm.at[:, pl.ds(bs, 128)], ti_v)
        for j in range(JV):
            sl = pl.ds(j * LANES, LANES)
            m1, i1 = m_v[sl], i_v[sl]
            for r in range(8):
                m2 = tm_v[r, sl]
                i2 = ti_v[r, sl]
                take2 = (m2 > m1) | ((m2 == m1) & (i2 < i1))
                m1 = jnp.where(take2, m2, m1)
                i1 = jnp.where(take2, i2, i1)
            m_v[sl] = m1
            i_v[sl] = i1
        g = pltpu.make_async_copy(tab_hbm.at[i_v], cast_v, gsem)
        g.start()
        g.wait()
        pltpu.sync_copy(cast_v, cast_hbm.at[pl.ds(bs, 128)])
        pltpu.sync_copy(m_v, maxp_hbm.at[pl.ds(bs, 128)])


def kernel(input, table_values):
    inpt = input.T
    sm, si = _sc_scan(inpt)
    tm, ti = _tc_scan(inpt)
    return _merge(sm, si, tm, ti, table_values)
